# Initial kernel scaffold; baseline (speedup 1.0000x reference)
import jax
import jax.numpy as jnp
from jax.experimental import pallas as pl


def kernel(x, pe_w, pe_b, pe_scale, pe_shift, l0_dw_w, l0_dw_b, l0_dw_scale, l0_dw_shift, l0_pw_w, l0_pw_b, l0_pw_scale, l0_pw_shift, l1_dw_w, l1_dw_b, l1_dw_scale, l1_dw_shift, l1_pw_w, l1_pw_b, l1_pw_scale, l1_pw_shift, l2_dw_w, l2_dw_b, l2_dw_scale, l2_dw_shift, l2_pw_w, l2_pw_b, l2_pw_scale, l2_pw_shift, l3_dw_w, l3_dw_b, l3_dw_scale, l3_dw_shift, l3_pw_w, l3_pw_b, l3_pw_scale, l3_pw_shift, l4_dw_w, l4_dw_b, l4_dw_scale, l4_dw_shift, l4_pw_w, l4_pw_b, l4_pw_scale, l4_pw_shift, l5_dw_w, l5_dw_b, l5_dw_scale, l5_dw_shift, l5_pw_w, l5_pw_b, l5_pw_scale, l5_pw_shift, l6_dw_w, l6_dw_b, l6_dw_scale, l6_dw_shift, l6_pw_w, l6_pw_b, l6_pw_scale, l6_pw_shift, l7_dw_w, l7_dw_b, l7_dw_scale, l7_dw_shift, l7_pw_w, l7_pw_b, l7_pw_scale, l7_pw_shift):
    raise NotImplementedError("write your pallas kernel here")


# speedup vs baseline: 1.9079x; 1.9079x over previous
import functools
import math

import jax
import jax.numpy as jnp
from jax.experimental import pallas as pl
from jax.experimental.pallas import tpu as pltpu

_INV_SQRT2 = 1.0 / math.sqrt(2.0)
_VMEM_LIMIT = 64 * 1024 * 1024


def _gelu(x):
    return 0.5 * x * (1.0 + jax.lax.erf(x * _INV_SQRT2))


def _convmixer_kernel(p_ref, pe_w_ref, pe_aux_ref,
                      dw_w_ref, dw_aux_ref, pw_w_ref, pw_aux_ref,
                      o_ref, xpad_ref, ybuf_ref, *, HP, WP, K, PAD, L):
    HW = HP * WP

    feat = jnp.dot(p_ref[0], pe_w_ref[...], preferred_element_type=jnp.float32)
    a = pe_aux_ref[...]
    feat = _gelu(feat + a[0]) * a[1] + a[2]

    xpad_ref[...] = jnp.zeros_like(xpad_ref)
    for r in range(HP):
        xpad_ref[r + PAD, pl.ds(PAD, WP), :] = feat[r * WP:(r + 1) * WP]

    def layer(l, _):
        w_all = dw_w_ref[l]
        aux = dw_aux_ref[l]

        def row(h, _):
            acc = xpad_ref[h, pl.ds(0, WP), :] * w_all[0]
            for t in range(1, K * K):
                i, j = divmod(t, K)
                acc = acc + xpad_ref[h + i, pl.ds(j, WP), :] * w_all[t]
            y = (_gelu(acc + aux[0]) * aux[1] + aux[2]
                 + xpad_ref[h + PAD, pl.ds(PAD, WP), :])
            ybuf_ref[pl.ds(h * WP, WP), :] = y.astype(jnp.bfloat16)
            return 0

        jax.lax.fori_loop(0, HP, row, 0)

        z = jnp.dot(ybuf_ref[...], pw_w_ref[l],
                    preferred_element_type=jnp.float32)
        paux = pw_aux_ref[l]
        z = _gelu(z + paux[0]) * paux[1] + paux[2]
        for r in range(HP):
            xpad_ref[r + PAD, pl.ds(PAD, WP), :] = z[r * WP:(r + 1) * WP]
        return 0

    jax.lax.fori_loop(0, L, layer, 0)

    interior = xpad_ref[pl.ds(PAD, HP), pl.ds(PAD, WP), :]
    o_ref[0, 0, :] = jnp.mean(interior, axis=(0, 1))


def _convmixer_fused(patches, pe_w, pe_aux, dw_w, dw_aux, pw_w, pw_aux,
                     *, HP, WP, K):
    n = patches.shape[0]
    cpp = patches.shape[2]
    d = pe_w.shape[1]
    L = dw_w.shape[0]
    pad = K // 2
    hw = HP * WP

    kern = functools.partial(_convmixer_kernel, HP=HP, WP=WP, K=K, PAD=pad, L=L)
    out = pl.pallas_call(
        kern,
        out_shape=jax.ShapeDtypeStruct((n, 1, d), jnp.float32),
        grid_spec=pltpu.PrefetchScalarGridSpec(
            num_scalar_prefetch=0,
            grid=(n,),
            in_specs=[
                pl.BlockSpec((1, hw, cpp), lambda b: (b, 0, 0)),
                pl.BlockSpec((cpp, d), lambda b: (0, 0)),
                pl.BlockSpec((3, d), lambda b: (0, 0)),
                pl.BlockSpec((L, K * K, d), lambda b: (0, 0, 0)),
                pl.BlockSpec((L, 3, d), lambda b: (0, 0, 0)),
                pl.BlockSpec((L, d, d), lambda b: (0, 0, 0)),
                pl.BlockSpec((L, 3, d), lambda b: (0, 0, 0)),
            ],
            out_specs=pl.BlockSpec((1, 1, d), lambda b: (b, 0, 0)),
            scratch_shapes=[
                pltpu.VMEM((HP + 2 * pad, WP + 2 * pad, d), jnp.float32),
                pltpu.VMEM((hw, d), jnp.bfloat16),
            ],
        ),
        compiler_params=pltpu.CompilerParams(
            dimension_semantics=("parallel",),
            vmem_limit_bytes=_VMEM_LIMIT),
        cost_estimate=pl.CostEstimate(
            flops=n * hw * d * (2 * cpp + L * (2 * K * K + 2 * d)),
            transcendentals=n * hw * d * (1 + 2 * L),
            bytes_accessed=n * hw * cpp * 2 + L * d * d * 2 + n * d * 4),
    )(patches, pe_w, pe_aux, dw_w, dw_aux, pw_w, pw_aux)
    return out.reshape(n, d)


def kernel(x, pe_w, pe_b, pe_scale, pe_shift, l0_dw_w, l0_dw_b, l0_dw_scale, l0_dw_shift, l0_pw_w, l0_pw_b, l0_pw_scale, l0_pw_shift, l1_dw_w, l1_dw_b, l1_dw_scale, l1_dw_shift, l1_pw_w, l1_pw_b, l1_pw_scale, l1_pw_shift, l2_dw_w, l2_dw_b, l2_dw_scale, l2_dw_shift, l2_pw_w, l2_pw_b, l2_pw_scale, l2_pw_shift, l3_dw_w, l3_dw_b, l3_dw_scale, l3_dw_shift, l3_pw_w, l3_pw_b, l3_pw_scale, l3_pw_shift, l4_dw_w, l4_dw_b, l4_dw_scale, l4_dw_shift, l4_pw_w, l4_pw_b, l4_pw_scale, l4_pw_shift, l5_dw_w, l5_dw_b, l5_dw_scale, l5_dw_shift, l5_pw_w, l5_pw_b, l5_pw_scale, l5_pw_shift, l6_dw_w, l6_dw_b, l6_dw_scale, l6_dw_shift, l6_pw_w, l6_pw_b, l6_pw_scale, l6_pw_shift, l7_dw_w, l7_dw_b, l7_dw_scale, l7_dw_shift, l7_pw_w, l7_pw_b, l7_pw_scale, l7_pw_shift):
    n, c, h, w = x.shape
    p = 7
    hp, wp = h // p, w // p
    d = pe_w.shape[1]
    kk = int(round(math.sqrt(l0_dw_w.shape[0])))

    patches = (x.reshape(n, c, hp, p, wp, p)
                .transpose(0, 2, 4, 1, 3, 5)
                .reshape(n, hp * wp, c * p * p)
                .astype(jnp.bfloat16))

    layers = [
        (l0_dw_w, l0_dw_b, l0_dw_scale, l0_dw_shift, l0_pw_w, l0_pw_b, l0_pw_scale, l0_pw_shift),
        (l1_dw_w, l1_dw_b, l1_dw_scale, l1_dw_shift, l1_pw_w, l1_pw_b, l1_pw_scale, l1_pw_shift),
        (l2_dw_w, l2_dw_b, l2_dw_scale, l2_dw_shift, l2_pw_w, l2_pw_b, l2_pw_scale, l2_pw_shift),
        (l3_dw_w, l3_dw_b, l3_dw_scale, l3_dw_shift, l3_pw_w, l3_pw_b, l3_pw_scale, l3_pw_shift),
        (l4_dw_w, l4_dw_b, l4_dw_scale, l4_dw_shift, l4_pw_w, l4_pw_b, l4_pw_scale, l4_pw_shift),
        (l5_dw_w, l5_dw_b, l5_dw_scale, l5_dw_shift, l5_pw_w, l5_pw_b, l5_pw_scale, l5_pw_shift),
        (l6_dw_w, l6_dw_b, l6_dw_scale, l6_dw_shift, l6_pw_w, l6_pw_b, l6_pw_scale, l6_pw_shift),
        (l7_dw_w, l7_dw_b, l7_dw_scale, l7_dw_shift, l7_pw_w, l7_pw_b, l7_pw_scale, l7_pw_shift),
    ]
    pe_aux = jnp.stack([pe_b, pe_scale, pe_shift])
    dw_w_all = jnp.stack([lw[0] for lw in layers])
    dw_aux = jnp.stack([jnp.stack([lw[1], lw[2], lw[3]]) for lw in layers])
    pw_w_all = jnp.stack([lw[4] for lw in layers]).astype(jnp.bfloat16)
    pw_aux = jnp.stack([jnp.stack([lw[5], lw[6], lw[7]]) for lw in layers])

    return _convmixer_fused(patches, pe_w.astype(jnp.bfloat16), pe_aux,
                            dw_w_all, dw_aux, pw_w_all, pw_aux,
                            HP=hp, WP=wp, K=kk)


# speedup vs baseline: 3.0838x; 1.6163x over previous
import functools
import math

import jax
import jax.numpy as jnp
from jax.experimental import pallas as pl
from jax.experimental.pallas import tpu as pltpu

_INV_SQRT2 = 1.0 / math.sqrt(2.0)
_VMEM_LIMIT = 64 * 1024 * 1024


def _gelu(x):
    return 0.5 * x * (1.0 + jax.lax.erf(x * _INV_SQRT2))


def _convmixer_kernel(p_ref, pe_w_ref, pe_aux_ref,
                      dw_w_ref, dw_aux_ref, pw_w_ref, pw_aux_ref,
                      o_ref, xpad_ref, ybuf_ref, s_ref, *, HP, WP, K, PAD, L):
    HW = HP * WP
    HPAD = HP + 2 * PAD

    feat = jnp.dot(p_ref[0], pe_w_ref[...], preferred_element_type=jnp.float32)
    a = pe_aux_ref[...]
    feat = _gelu(feat + a[0]) * a[1] + a[2]

    xpad_ref[...] = jnp.zeros_like(xpad_ref)
    for r in range(HP):
        xpad_ref[r + PAD, pl.ds(PAD, WP), :] = feat[r * WP:(r + 1) * WP]

    def layer(l, _):
        w_all = dw_w_ref[l]
        aux = dw_aux_ref[l]

        def stage(r, _):
            packed = xpad_ref[r].astype(jnp.bfloat16)
            for j in range(K):
                s_ref[j, r] = packed[j:j + WP, :]
            return 0

        jax.lax.fori_loop(0, HPAD, stage, 0)

        def row(h, _):
            acc = None
            for i in range(K):
                g = s_ref[0, h + i] * w_all[i * K]
                for j in range(1, K):
                    g = g + s_ref[j, h + i] * w_all[i * K + j]
                acc = g.astype(jnp.float32) if acc is None else acc + g.astype(jnp.float32)
            y = (_gelu(acc + aux[0]) * aux[1] + aux[2]
                 + xpad_ref[h + PAD, pl.ds(PAD, WP), :])
            ybuf_ref[pl.ds(h * WP, WP), :] = y.astype(jnp.bfloat16)
            return 0

        jax.lax.fori_loop(0, HP, row, 0)

        z = jnp.dot(ybuf_ref[...], pw_w_ref[l],
                    preferred_element_type=jnp.float32)
        paux = pw_aux_ref[l]
        z = _gelu(z + paux[0]) * paux[1] + paux[2]
        for r in range(HP):
            xpad_ref[r + PAD, pl.ds(PAD, WP), :] = z[r * WP:(r + 1) * WP]
        return 0

    jax.lax.fori_loop(0, L, layer, 0)

    interior = xpad_ref[pl.ds(PAD, HP), pl.ds(PAD, WP), :]
    o_ref[0, 0, :] = jnp.mean(interior, axis=(0, 1))


def _convmixer_fused(patches, pe_w, pe_aux, dw_w, dw_aux, pw_w, pw_aux,
                     *, HP, WP, K):
    n = patches.shape[0]
    cpp = patches.shape[2]
    d = pe_w.shape[1]
    L = dw_w.shape[0]
    pad = K // 2
    hw = HP * WP

    kern = functools.partial(_convmixer_kernel, HP=HP, WP=WP, K=K, PAD=pad, L=L)
    out = pl.pallas_call(
        kern,
        out_shape=jax.ShapeDtypeStruct((n, 1, d), jnp.float32),
        grid_spec=pltpu.PrefetchScalarGridSpec(
            num_scalar_prefetch=0,
            grid=(n,),
            in_specs=[
                pl.BlockSpec((1, hw, cpp), lambda b: (b, 0, 0)),
                pl.BlockSpec((cpp, d), lambda b: (0, 0)),
                pl.BlockSpec((3, d), lambda b: (0, 0)),
                pl.BlockSpec((L, K * K, d), lambda b: (0, 0, 0)),
                pl.BlockSpec((L, 3, d), lambda b: (0, 0, 0)),
                pl.BlockSpec((L, d, d), lambda b: (0, 0, 0)),
                pl.BlockSpec((L, 3, d), lambda b: (0, 0, 0)),
            ],
            out_specs=pl.BlockSpec((1, 1, d), lambda b: (b, 0, 0)),
            scratch_shapes=[
                pltpu.VMEM((HP + 2 * pad, WP + 2 * pad, d), jnp.float32),
                pltpu.VMEM((hw, d), jnp.bfloat16),
                pltpu.VMEM((K, HP + 2 * pad, WP, d), jnp.bfloat16),
            ],
        ),
        compiler_params=pltpu.CompilerParams(
            dimension_semantics=("parallel",),
            vmem_limit_bytes=_VMEM_LIMIT),
        cost_estimate=pl.CostEstimate(
            flops=n * hw * d * (2 * cpp + L * (2 * K * K + 2 * d)),
            transcendentals=n * hw * d * (1 + 2 * L),
            bytes_accessed=n * hw * cpp * 2 + L * d * d * 2 + n * d * 4),
    )(patches, pe_w, pe_aux, dw_w, dw_aux, pw_w, pw_aux)
    return out.reshape(n, d)


def kernel(x, pe_w, pe_b, pe_scale, pe_shift, l0_dw_w, l0_dw_b, l0_dw_scale, l0_dw_shift, l0_pw_w, l0_pw_b, l0_pw_scale, l0_pw_shift, l1_dw_w, l1_dw_b, l1_dw_scale, l1_dw_shift, l1_pw_w, l1_pw_b, l1_pw_scale, l1_pw_shift, l2_dw_w, l2_dw_b, l2_dw_scale, l2_dw_shift, l2_pw_w, l2_pw_b, l2_pw_scale, l2_pw_shift, l3_dw_w, l3_dw_b, l3_dw_scale, l3_dw_shift, l3_pw_w, l3_pw_b, l3_pw_scale, l3_pw_shift, l4_dw_w, l4_dw_b, l4_dw_scale, l4_dw_shift, l4_pw_w, l4_pw_b, l4_pw_scale, l4_pw_shift, l5_dw_w, l5_dw_b, l5_dw_scale, l5_dw_shift, l5_pw_w, l5_pw_b, l5_pw_scale, l5_pw_shift, l6_dw_w, l6_dw_b, l6_dw_scale, l6_dw_shift, l6_pw_w, l6_pw_b, l6_pw_scale, l6_pw_shift, l7_dw_w, l7_dw_b, l7_dw_scale, l7_dw_shift, l7_pw_w, l7_pw_b, l7_pw_scale, l7_pw_shift):
    n, c, h, w = x.shape
    p = 7
    hp, wp = h // p, w // p
    d = pe_w.shape[1]
    kk = int(round(math.sqrt(l0_dw_w.shape[0])))

    patches = (x.reshape(n, c, hp, p, wp, p)
                .transpose(0, 2, 4, 1, 3, 5)
                .reshape(n, hp * wp, c * p * p)
                .astype(jnp.bfloat16))

    layers = [
        (l0_dw_w, l0_dw_b, l0_dw_scale, l0_dw_shift, l0_pw_w, l0_pw_b, l0_pw_scale, l0_pw_shift),
        (l1_dw_w, l1_dw_b, l1_dw_scale, l1_dw_shift, l1_pw_w, l1_pw_b, l1_pw_scale, l1_pw_shift),
        (l2_dw_w, l2_dw_b, l2_dw_scale, l2_dw_shift, l2_pw_w, l2_pw_b, l2_pw_scale, l2_pw_shift),
        (l3_dw_w, l3_dw_b, l3_dw_scale, l3_dw_shift, l3_pw_w, l3_pw_b, l3_pw_scale, l3_pw_shift),
        (l4_dw_w, l4_dw_b, l4_dw_scale, l4_dw_shift, l4_pw_w, l4_pw_b, l4_pw_scale, l4_pw_shift),
        (l5_dw_w, l5_dw_b, l5_dw_scale, l5_dw_shift, l5_pw_w, l5_pw_b, l5_pw_scale, l5_pw_shift),
        (l6_dw_w, l6_dw_b, l6_dw_scale, l6_dw_shift, l6_pw_w, l6_pw_b, l6_pw_scale, l6_pw_shift),
        (l7_dw_w, l7_dw_b, l7_dw_scale, l7_dw_shift, l7_pw_w, l7_pw_b, l7_pw_scale, l7_pw_shift),
    ]
    pe_aux = jnp.stack([pe_b, pe_scale, pe_shift])
    dw_w_all = jnp.stack([lw[0] for lw in layers]).astype(jnp.bfloat16)
    dw_aux = jnp.stack([jnp.stack([lw[1], lw[2], lw[3]]) for lw in layers])
    pw_w_all = jnp.stack([lw[4] for lw in layers]).astype(jnp.bfloat16)
    pw_aux = jnp.stack([jnp.stack([lw[5], lw[6], lw[7]]) for lw in layers])

    return _convmixer_fused(patches, pe_w.astype(jnp.bfloat16), pe_aux,
                            dw_w_all, dw_aux, pw_w_all, pw_aux,
                            HP=hp, WP=wp, K=kk)


# speedup vs baseline: 3.8670x; 1.2540x over previous
import functools
import math

import jax
import jax.numpy as jnp
from jax.experimental import pallas as pl
from jax.experimental.pallas import tpu as pltpu

_INV_SQRT2 = 1.0 / math.sqrt(2.0)
_VMEM_LIMIT = 64 * 1024 * 1024


def _gelu(x):
    return 0.5 * x * (1.0 + jax.lax.erf(x * _INV_SQRT2))


def _convmixer_kernel(p_ref, pe_w_ref, pe_aux_ref,
                      dw_w_ref, dw_aux_ref, pw_w_ref, pw_aux_ref,
                      o_ref, xpad_ref, ybuf_ref, s_ref, *, HP, WP, K, PAD, L):
    HW = HP * WP
    HPAD = HP + 2 * PAD
    CB = 16
    HH = WP // 2

    feat = jnp.dot(p_ref[0], pe_w_ref[...], preferred_element_type=jnp.float32)
    a = pe_aux_ref[...]
    feat = _gelu(feat + a[0]) * a[1] + a[2]

    xpad_ref[...] = jnp.zeros_like(xpad_ref)
    for r in range(HP):
        xpad_ref[r + PAD, pl.ds(CB, WP), :] = (
            feat[r * WP:(r + 1) * WP].astype(jnp.bfloat16))

    def layer(l, _):
        aux = dw_aux_ref[l]

        def stage(r, _):
            packed = xpad_ref[r]
            for j in range(K):
                s_ref[j, r] = packed[CB - PAD + j:CB - PAD + j + WP, :]
            return 0

        jax.lax.fori_loop(0, HPAD, stage, 0)

        def row(h, _):
            accA = accB = None
            for i in range(K):
                wt = dw_w_ref[l, i * K]
                gA = s_ref[0, h + i, 0:HH, :] * wt
                gB = s_ref[0, h + i, HH:WP, :] * wt
                for j in range(1, K):
                    wt = dw_w_ref[l, i * K + j]
                    gA = gA + s_ref[j, h + i, 0:HH, :] * wt
                    gB = gB + s_ref[j, h + i, HH:WP, :] * wt
                if accA is None:
                    accA = gA.astype(jnp.float32)
                    accB = gB.astype(jnp.float32)
                else:
                    accA = accA + gA.astype(jnp.float32)
                    accB = accB + gB.astype(jnp.float32)
            x0A = xpad_ref[h + PAD, pl.ds(CB, HH), :].astype(jnp.float32)
            x0B = xpad_ref[h + PAD, pl.ds(CB + HH, HH), :].astype(jnp.float32)
            yA = _gelu(accA + aux[0]) * aux[1] + aux[2] + x0A
            yB = _gelu(accB + aux[0]) * aux[1] + aux[2] + x0B
            ybuf_ref[pl.ds(h * WP, HH), :] = yA.astype(jnp.bfloat16)
            ybuf_ref[pl.ds(h * WP + HH, HH), :] = yB.astype(jnp.bfloat16)
            return 0

        jax.lax.fori_loop(0, HP, row, 0)

        z = jnp.dot(ybuf_ref[...], pw_w_ref[l],
                    preferred_element_type=jnp.float32)
        paux = pw_aux_ref[l]
        z = _gelu(z + paux[0]) * paux[1] + paux[2]
        for r in range(HP):
            xpad_ref[r + PAD, pl.ds(CB, WP), :] = (
                z[r * WP:(r + 1) * WP].astype(jnp.bfloat16))
        return 0

    jax.lax.fori_loop(0, L, layer, 0)

    interior = xpad_ref[pl.ds(PAD, HP), pl.ds(CB, WP), :].astype(jnp.float32)
    o_ref[0, 0, :] = jnp.mean(interior, axis=(0, 1))


def _convmixer_fused(patches, pe_w, pe_aux, dw_w, dw_aux, pw_w, pw_aux,
                     *, HP, WP, K):
    n = patches.shape[0]
    cpp = patches.shape[2]
    d = pe_w.shape[1]
    L = dw_w.shape[0]
    pad = K // 2
    hw = HP * WP

    kern = functools.partial(_convmixer_kernel, HP=HP, WP=WP, K=K, PAD=pad, L=L)
    out = pl.pallas_call(
        kern,
        out_shape=jax.ShapeDtypeStruct((n, 1, d), jnp.float32),
        grid_spec=pltpu.PrefetchScalarGridSpec(
            num_scalar_prefetch=0,
            grid=(n,),
            in_specs=[
                pl.BlockSpec((1, hw, cpp), lambda b: (b, 0, 0)),
                pl.BlockSpec((cpp, d), lambda b: (0, 0)),
                pl.BlockSpec((3, d), lambda b: (0, 0)),
                pl.BlockSpec((L, K * K, 16, d), lambda b: (0, 0, 0, 0)),
                pl.BlockSpec((L, 3, d), lambda b: (0, 0, 0)),
                pl.BlockSpec((L, d, d), lambda b: (0, 0, 0)),
                pl.BlockSpec((L, 3, d), lambda b: (0, 0, 0)),
            ],
            out_specs=pl.BlockSpec((1, 1, d), lambda b: (b, 0, 0)),
            scratch_shapes=[
                pltpu.VMEM((HP + 2 * pad, 64, d), jnp.bfloat16),
                pltpu.VMEM((hw, d), jnp.bfloat16),
                pltpu.VMEM((K, HP + 2 * pad, WP, d), jnp.bfloat16),
            ],
        ),
        compiler_params=pltpu.CompilerParams(
            dimension_semantics=("parallel",),
            vmem_limit_bytes=_VMEM_LIMIT),
        cost_estimate=pl.CostEstimate(
            flops=n * hw * d * (2 * cpp + L * (2 * K * K + 2 * d)),
            transcendentals=n * hw * d * (1 + 2 * L),
            bytes_accessed=n * hw * cpp * 2 + L * d * d * 2 + n * d * 4),
    )(patches, pe_w, pe_aux, dw_w, dw_aux, pw_w, pw_aux)
    return out.reshape(n, d)


def kernel(x, pe_w, pe_b, pe_scale, pe_shift, l0_dw_w, l0_dw_b, l0_dw_scale, l0_dw_shift, l0_pw_w, l0_pw_b, l0_pw_scale, l0_pw_shift, l1_dw_w, l1_dw_b, l1_dw_scale, l1_dw_shift, l1_pw_w, l1_pw_b, l1_pw_scale, l1_pw_shift, l2_dw_w, l2_dw_b, l2_dw_scale, l2_dw_shift, l2_pw_w, l2_pw_b, l2_pw_scale, l2_pw_shift, l3_dw_w, l3_dw_b, l3_dw_scale, l3_dw_shift, l3_pw_w, l3_pw_b, l3_pw_scale, l3_pw_shift, l4_dw_w, l4_dw_b, l4_dw_scale, l4_dw_shift, l4_pw_w, l4_pw_b, l4_pw_scale, l4_pw_shift, l5_dw_w, l5_dw_b, l5_dw_scale, l5_dw_shift, l5_pw_w, l5_pw_b, l5_pw_scale, l5_pw_shift, l6_dw_w, l6_dw_b, l6_dw_scale, l6_dw_shift, l6_pw_w, l6_pw_b, l6_pw_scale, l6_pw_shift, l7_dw_w, l7_dw_b, l7_dw_scale, l7_dw_shift, l7_pw_w, l7_pw_b, l7_pw_scale, l7_pw_shift):
    n, c, h, w = x.shape
    p = 7
    hp, wp = h // p, w // p
    d = pe_w.shape[1]
    kk = int(round(math.sqrt(l0_dw_w.shape[0])))

    patches = (x.reshape(n, c, hp, p, wp, p)
                .transpose(0, 2, 4, 1, 3, 5)
                .reshape(n, hp * wp, c * p * p)
                .astype(jnp.bfloat16))

    layers = [
        (l0_dw_w, l0_dw_b, l0_dw_scale, l0_dw_shift, l0_pw_w, l0_pw_b, l0_pw_scale, l0_pw_shift),
        (l1_dw_w, l1_dw_b, l1_dw_scale, l1_dw_shift, l1_pw_w, l1_pw_b, l1_pw_scale, l1_pw_shift),
        (l2_dw_w, l2_dw_b, l2_dw_scale, l2_dw_shift, l2_pw_w, l2_pw_b, l2_pw_scale, l2_pw_shift),
        (l3_dw_w, l3_dw_b, l3_dw_scale, l3_dw_shift, l3_pw_w, l3_pw_b, l3_pw_scale, l3_pw_shift),
        (l4_dw_w, l4_dw_b, l4_dw_scale, l4_dw_shift, l4_pw_w, l4_pw_b, l4_pw_scale, l4_pw_shift),
        (l5_dw_w, l5_dw_b, l5_dw_scale, l5_dw_shift, l5_pw_w, l5_pw_b, l5_pw_scale, l5_pw_shift),
        (l6_dw_w, l6_dw_b, l6_dw_scale, l6_dw_shift, l6_pw_w, l6_pw_b, l6_pw_scale, l6_pw_shift),
        (l7_dw_w, l7_dw_b, l7_dw_scale, l7_dw_shift, l7_pw_w, l7_pw_b, l7_pw_scale, l7_pw_shift),
    ]
    pe_aux = jnp.stack([pe_b, pe_scale, pe_shift])
    dw_w_all = jnp.stack([lw[0] for lw in layers]).astype(jnp.bfloat16)
    dw_w_rep = jnp.broadcast_to(dw_w_all[:, :, None, :],
                                (len(layers), dw_w_all.shape[1], 16, d))
    dw_aux = jnp.stack([jnp.stack([lw[1], lw[2], lw[3]]) for lw in layers])
    pw_w_all = jnp.stack([lw[4] for lw in layers]).astype(jnp.bfloat16)
    pw_aux = jnp.stack([jnp.stack([lw[5], lw[6], lw[7]]) for lw in layers])

    return _convmixer_fused(patches, pe_w.astype(jnp.bfloat16), pe_aux,
                            dw_w_rep, dw_aux, pw_w_all, pw_aux,
                            HP=hp, WP=wp, K=kk)


# speedup vs baseline: 3.9576x; 1.0234x over previous
import functools
import math

import jax
import jax.numpy as jnp
from jax.experimental import pallas as pl
from jax.experimental.pallas import tpu as pltpu

_INV_SQRT2 = 1.0 / math.sqrt(2.0)
_VMEM_LIMIT = 64 * 1024 * 1024


def _gelu(x):
    return 0.5 * x * (1.0 + jax.lax.erf(x * _INV_SQRT2))


def _convmixer_kernel(p_ref, pe_w_ref, pe_aux_ref,
                      dw_w_ref, dw_aux_ref, pw_w_ref, pw_aux_ref,
                      o_ref, xpad_ref, ybuf_ref, s_ref, *, HP, WP, K, PAD, L):
    HW = HP * WP
    HPAD = HP + 2 * PAD
    CB = 16
    HH = WP // 2

    feat = jnp.dot(p_ref[0], pe_w_ref[...], preferred_element_type=jnp.float32)
    a = pe_aux_ref[...]
    feat = _gelu(feat + a[0]) * a[1] + a[2]

    xpad_ref[...] = jnp.zeros_like(xpad_ref)
    for r in range(HP):
        xpad_ref[r + PAD, pl.ds(CB, WP), :] = (
            feat[r * WP:(r + 1) * WP].astype(jnp.bfloat16))

    def layer(l, _):
        aux = dw_aux_ref[l]

        def stage(r, _):
            packed = xpad_ref[r]
            for j in range(K):
                s_ref[j, r] = packed[CB - PAD + j:CB - PAD + j + WP, :]
            return 0

        jax.lax.fori_loop(0, HPAD, stage, 0)

        def row(h, _):
            accA = accB = None
            for i in range(K):
                for j in range(K):
                    wt = dw_w_ref[l, i * K + j]
                    if j == 0 and i % 3 == 0:
                        gA = s_ref[0, h + i, 0:HH, :] * wt
                        gB = s_ref[0, h + i, HH:WP, :] * wt
                    else:
                        gA = gA + s_ref[j, h + i, 0:HH, :] * wt
                        gB = gB + s_ref[j, h + i, HH:WP, :] * wt
                if i % 3 == 2:
                    if accA is None:
                        accA = gA.astype(jnp.float32)
                        accB = gB.astype(jnp.float32)
                    else:
                        accA = accA + gA.astype(jnp.float32)
                        accB = accB + gB.astype(jnp.float32)
            x0A = s_ref[PAD, h + PAD, 0:HH, :].astype(jnp.float32)
            x0B = s_ref[PAD, h + PAD, HH:WP, :].astype(jnp.float32)
            hA = _gelu(accA + aux[0])
            hB = _gelu(accB + aux[0])
            yA = hA * aux[1] + (aux[2] + x0A)
            yB = hB * aux[1] + (aux[2] + x0B)
            ybuf_ref[pl.ds(h * WP, HH), :] = yA.astype(jnp.bfloat16)
            ybuf_ref[pl.ds(h * WP + HH, HH), :] = yB.astype(jnp.bfloat16)
            return 0

        jax.lax.fori_loop(0, HP, row, 0)

        z = jnp.dot(ybuf_ref[...], pw_w_ref[l],
                    preferred_element_type=jnp.float32)
        paux = pw_aux_ref[l]
        z = _gelu(z + paux[0]) * paux[1] + paux[2]
        for r in range(HP):
            xpad_ref[r + PAD, pl.ds(CB, WP), :] = (
                z[r * WP:(r + 1) * WP].astype(jnp.bfloat16))
        return 0

    jax.lax.fori_loop(0, L, layer, 0)

    interior = xpad_ref[pl.ds(PAD, HP), pl.ds(CB, WP), :].astype(jnp.float32)
    o_ref[0, 0, :] = jnp.mean(interior, axis=(0, 1))


def _convmixer_fused(patches, pe_w, pe_aux, dw_w, dw_aux, pw_w, pw_aux,
                     *, HP, WP, K):
    n = patches.shape[0]
    cpp = patches.shape[2]
    d = pe_w.shape[1]
    L = dw_w.shape[0]
    pad = K // 2
    hw = HP * WP

    kern = functools.partial(_convmixer_kernel, HP=HP, WP=WP, K=K, PAD=pad, L=L)
    out = pl.pallas_call(
        kern,
        out_shape=jax.ShapeDtypeStruct((n, 1, d), jnp.float32),
        grid_spec=pltpu.PrefetchScalarGridSpec(
            num_scalar_prefetch=0,
            grid=(n,),
            in_specs=[
                pl.BlockSpec((1, hw, cpp), lambda b: (b, 0, 0)),
                pl.BlockSpec((cpp, d), lambda b: (0, 0)),
                pl.BlockSpec((3, d), lambda b: (0, 0)),
                pl.BlockSpec((L, K * K, 16, d), lambda b: (0, 0, 0, 0)),
                pl.BlockSpec((L, 3, d), lambda b: (0, 0, 0)),
                pl.BlockSpec((L, d, d), lambda b: (0, 0, 0)),
                pl.BlockSpec((L, 3, d), lambda b: (0, 0, 0)),
            ],
            out_specs=pl.BlockSpec((1, 1, d), lambda b: (b, 0, 0)),
            scratch_shapes=[
                pltpu.VMEM((HP + 2 * pad, 64, d), jnp.bfloat16),
                pltpu.VMEM((hw, d), jnp.bfloat16),
                pltpu.VMEM((K, HP + 2 * pad, WP, d), jnp.bfloat16),
            ],
        ),
        compiler_params=pltpu.CompilerParams(
            dimension_semantics=("parallel",),
            vmem_limit_bytes=_VMEM_LIMIT),
        cost_estimate=pl.CostEstimate(
            flops=n * hw * d * (2 * cpp + L * (2 * K * K + 2 * d)),
            transcendentals=n * hw * d * (1 + 2 * L),
            bytes_accessed=n * hw * cpp * 2 + L * d * d * 2 + n * d * 4),
    )(patches, pe_w, pe_aux, dw_w, dw_aux, pw_w, pw_aux)
    return out.reshape(n, d)


def kernel(x, pe_w, pe_b, pe_scale, pe_shift, l0_dw_w, l0_dw_b, l0_dw_scale, l0_dw_shift, l0_pw_w, l0_pw_b, l0_pw_scale, l0_pw_shift, l1_dw_w, l1_dw_b, l1_dw_scale, l1_dw_shift, l1_pw_w, l1_pw_b, l1_pw_scale, l1_pw_shift, l2_dw_w, l2_dw_b, l2_dw_scale, l2_dw_shift, l2_pw_w, l2_pw_b, l2_pw_scale, l2_pw_shift, l3_dw_w, l3_dw_b, l3_dw_scale, l3_dw_shift, l3_pw_w, l3_pw_b, l3_pw_scale, l3_pw_shift, l4_dw_w, l4_dw_b, l4_dw_scale, l4_dw_shift, l4_pw_w, l4_pw_b, l4_pw_scale, l4_pw_shift, l5_dw_w, l5_dw_b, l5_dw_scale, l5_dw_shift, l5_pw_w, l5_pw_b, l5_pw_scale, l5_pw_shift, l6_dw_w, l6_dw_b, l6_dw_scale, l6_dw_shift, l6_pw_w, l6_pw_b, l6_pw_scale, l6_pw_shift, l7_dw_w, l7_dw_b, l7_dw_scale, l7_dw_shift, l7_pw_w, l7_pw_b, l7_pw_scale, l7_pw_shift):
    n, c, h, w = x.shape
    p = 7
    hp, wp = h // p, w // p
    d = pe_w.shape[1]
    kk = int(round(math.sqrt(l0_dw_w.shape[0])))

    patches = (x.reshape(n, c, hp, p, wp, p)
                .transpose(0, 2, 4, 1, 3, 5)
                .reshape(n, hp * wp, c * p * p)
                .astype(jnp.bfloat16))

    layers = [
        (l0_dw_w, l0_dw_b, l0_dw_scale, l0_dw_shift, l0_pw_w, l0_pw_b, l0_pw_scale, l0_pw_shift),
        (l1_dw_w, l1_dw_b, l1_dw_scale, l1_dw_shift, l1_pw_w, l1_pw_b, l1_pw_scale, l1_pw_shift),
        (l2_dw_w, l2_dw_b, l2_dw_scale, l2_dw_shift, l2_pw_w, l2_pw_b, l2_pw_scale, l2_pw_shift),
        (l3_dw_w, l3_dw_b, l3_dw_scale, l3_dw_shift, l3_pw_w, l3_pw_b, l3_pw_scale, l3_pw_shift),
        (l4_dw_w, l4_dw_b, l4_dw_scale, l4_dw_shift, l4_pw_w, l4_pw_b, l4_pw_scale, l4_pw_shift),
        (l5_dw_w, l5_dw_b, l5_dw_scale, l5_dw_shift, l5_pw_w, l5_pw_b, l5_pw_scale, l5_pw_shift),
        (l6_dw_w, l6_dw_b, l6_dw_scale, l6_dw_shift, l6_pw_w, l6_pw_b, l6_pw_scale, l6_pw_shift),
        (l7_dw_w, l7_dw_b, l7_dw_scale, l7_dw_shift, l7_pw_w, l7_pw_b, l7_pw_scale, l7_pw_shift),
    ]
    pe_aux = jnp.stack([pe_b, pe_scale, pe_shift])
    dw_w_all = jnp.stack([lw[0] for lw in layers]).astype(jnp.bfloat16)
    dw_w_rep = jnp.broadcast_to(dw_w_all[:, :, None, :],
                                (len(layers), dw_w_all.shape[1], 16, d))
    dw_aux = jnp.stack([jnp.stack([lw[1], lw[2], lw[3]]) for lw in layers])
    pw_w_all = jnp.stack([lw[4] for lw in layers]).astype(jnp.bfloat16)
    pw_aux = jnp.stack([jnp.stack([lw[5], lw[6], lw[7]]) for lw in layers])

    return _convmixer_fused(patches, pe_w.astype(jnp.bfloat16), pe_aux,
                            dw_w_rep, dw_aux, pw_w_all, pw_aux,
                            HP=hp, WP=wp, K=kk)


# speedup vs baseline: 4.0110x; 1.0135x over previous
import functools
import math

import jax
import jax.numpy as jnp
from jax.experimental import pallas as pl
from jax.experimental.pallas import tpu as pltpu

_INV_SQRT2 = 1.0 / math.sqrt(2.0)
_VMEM_LIMIT = 64 * 1024 * 1024


def _gelu(x):
    return 0.5 * x * (1.0 + jax.lax.erf(x * _INV_SQRT2))


def _convmixer_kernel(p_ref, pe_w_ref, pe_aux_ref,
                      dw_w_ref, dw_aux_ref, pw_w_ref, pw_aux_ref,
                      o_ref, ybuf_ref, s_ref, *, HP, WP, K, PAD, L):
    HW = HP * WP
    HH = WP // 2
    D = o_ref.shape[-1]

    def write_shifts(p, row_bf16):
        zc = jnp.zeros((PAD, D), jnp.bfloat16)
        padded = jnp.concatenate([zc, row_bf16, zc], axis=0)
        for j in range(K):
            s_ref[j, p] = padded[j:j + WP, :]

    zrow = jnp.zeros((WP, D), jnp.bfloat16)
    for j in range(K):
        for r in range(PAD):
            s_ref[j, r] = zrow
            s_ref[j, HP + PAD + r] = zrow

    feat = jnp.dot(p_ref[0], pe_w_ref[...], preferred_element_type=jnp.float32)
    a = pe_aux_ref[...]
    feat = _gelu(feat + a[0]) * a[1] + a[2]
    for r in range(HP):
        write_shifts(r + PAD, feat[r * WP:(r + 1) * WP].astype(jnp.bfloat16))

    def layer(l, _):
        aux = dw_aux_ref[l]

        def row(h, _):
            accA = accB = None
            for i in range(K):
                for j in range(K):
                    wt = dw_w_ref[l, i * K + j]
                    if j == 0 and i % 3 == 0:
                        gA = s_ref[0, h + i, 0:HH, :] * wt
                        gB = s_ref[0, h + i, HH:WP, :] * wt
                    else:
                        gA = gA + s_ref[j, h + i, 0:HH, :] * wt
                        gB = gB + s_ref[j, h + i, HH:WP, :] * wt
                if i % 3 == 2:
                    if accA is None:
                        accA = gA.astype(jnp.float32)
                        accB = gB.astype(jnp.float32)
                    else:
                        accA = accA + gA.astype(jnp.float32)
                        accB = accB + gB.astype(jnp.float32)
            x0A = s_ref[PAD, h + PAD, 0:HH, :].astype(jnp.float32)
            x0B = s_ref[PAD, h + PAD, HH:WP, :].astype(jnp.float32)
            hA = _gelu(accA + aux[0])
            hB = _gelu(accB + aux[0])
            yA = hA * aux[1] + (aux[2] + x0A)
            yB = hB * aux[1] + (aux[2] + x0B)
            ybuf_ref[pl.ds(h * WP, HH), :] = yA.astype(jnp.bfloat16)
            ybuf_ref[pl.ds(h * WP + HH, HH), :] = yB.astype(jnp.bfloat16)
            return 0

        jax.lax.fori_loop(0, HP, row, 0)

        z = jnp.dot(ybuf_ref[...], pw_w_ref[l],
                    preferred_element_type=jnp.float32)
        paux = pw_aux_ref[l]
        for r in range(HP):
            zr = z[r * WP:(r + 1) * WP]
            zr = _gelu(zr + paux[0]) * paux[1] + paux[2]
            write_shifts(r + PAD, zr.astype(jnp.bfloat16))
        return 0

    jax.lax.fori_loop(0, L, layer, 0)

    interior = s_ref[PAD, pl.ds(PAD, HP), :, :].astype(jnp.float32)
    o_ref[0, 0, :] = jnp.mean(interior, axis=(0, 1))


def _convmixer_fused(patches, pe_w, pe_aux, dw_w, dw_aux, pw_w, pw_aux,
                     *, HP, WP, K):
    n = patches.shape[0]
    cpp = patches.shape[2]
    d = pe_w.shape[1]
    L = dw_w.shape[0]
    pad = K // 2
    hw = HP * WP

    kern = functools.partial(_convmixer_kernel, HP=HP, WP=WP, K=K, PAD=pad, L=L)
    out = pl.pallas_call(
        kern,
        out_shape=jax.ShapeDtypeStruct((n, 1, d), jnp.float32),
        grid_spec=pltpu.PrefetchScalarGridSpec(
            num_scalar_prefetch=0,
            grid=(n,),
            in_specs=[
                pl.BlockSpec((1, hw, cpp), lambda b: (b, 0, 0)),
                pl.BlockSpec((cpp, d), lambda b: (0, 0)),
                pl.BlockSpec((3, d), lambda b: (0, 0)),
                pl.BlockSpec((L, K * K, 16, d), lambda b: (0, 0, 0, 0)),
                pl.BlockSpec((L, 3, d), lambda b: (0, 0, 0)),
                pl.BlockSpec((L, d, d), lambda b: (0, 0, 0)),
                pl.BlockSpec((L, 3, d), lambda b: (0, 0, 0)),
            ],
            out_specs=pl.BlockSpec((1, 1, d), lambda b: (b, 0, 0)),
            scratch_shapes=[
                pltpu.VMEM((hw, d), jnp.bfloat16),
                pltpu.VMEM((K, HP + 2 * pad, WP, d), jnp.bfloat16),
            ],
        ),
        compiler_params=pltpu.CompilerParams(
            dimension_semantics=("parallel",),
            vmem_limit_bytes=_VMEM_LIMIT),
        cost_estimate=pl.CostEstimate(
            flops=n * hw * d * (2 * cpp + L * (2 * K * K + 2 * d)),
            transcendentals=n * hw * d * (1 + 2 * L),
            bytes_accessed=n * hw * cpp * 2 + L * d * d * 2 + n * d * 4),
    )(patches, pe_w, pe_aux, dw_w, dw_aux, pw_w, pw_aux)
    return out.reshape(n, d)


def kernel(x, pe_w, pe_b, pe_scale, pe_shift, l0_dw_w, l0_dw_b, l0_dw_scale, l0_dw_shift, l0_pw_w, l0_pw_b, l0_pw_scale, l0_pw_shift, l1_dw_w, l1_dw_b, l1_dw_scale, l1_dw_shift, l1_pw_w, l1_pw_b, l1_pw_scale, l1_pw_shift, l2_dw_w, l2_dw_b, l2_dw_scale, l2_dw_shift, l2_pw_w, l2_pw_b, l2_pw_scale, l2_pw_shift, l3_dw_w, l3_dw_b, l3_dw_scale, l3_dw_shift, l3_pw_w, l3_pw_b, l3_pw_scale, l3_pw_shift, l4_dw_w, l4_dw_b, l4_dw_scale, l4_dw_shift, l4_pw_w, l4_pw_b, l4_pw_scale, l4_pw_shift, l5_dw_w, l5_dw_b, l5_dw_scale, l5_dw_shift, l5_pw_w, l5_pw_b, l5_pw_scale, l5_pw_shift, l6_dw_w, l6_dw_b, l6_dw_scale, l6_dw_shift, l6_pw_w, l6_pw_b, l6_pw_scale, l6_pw_shift, l7_dw_w, l7_dw_b, l7_dw_scale, l7_dw_shift, l7_pw_w, l7_pw_b, l7_pw_scale, l7_pw_shift):
    n, c, h, w = x.shape
    p = 7
    hp, wp = h // p, w // p
    d = pe_w.shape[1]
    kk = int(round(math.sqrt(l0_dw_w.shape[0])))

    patches = (x.reshape(n, c, hp, p, wp, p)
                .transpose(0, 2, 4, 1, 3, 5)
                .reshape(n, hp * wp, c * p * p)
                .astype(jnp.bfloat16))

    layers = [
        (l0_dw_w, l0_dw_b, l0_dw_scale, l0_dw_shift, l0_pw_w, l0_pw_b, l0_pw_scale, l0_pw_shift),
        (l1_dw_w, l1_dw_b, l1_dw_scale, l1_dw_shift, l1_pw_w, l1_pw_b, l1_pw_scale, l1_pw_shift),
        (l2_dw_w, l2_dw_b, l2_dw_scale, l2_dw_shift, l2_pw_w, l2_pw_b, l2_pw_scale, l2_pw_shift),
        (l3_dw_w, l3_dw_b, l3_dw_scale, l3_dw_shift, l3_pw_w, l3_pw_b, l3_pw_scale, l3_pw_shift),
        (l4_dw_w, l4_dw_b, l4_dw_scale, l4_dw_shift, l4_pw_w, l4_pw_b, l4_pw_scale, l4_pw_shift),
        (l5_dw_w, l5_dw_b, l5_dw_scale, l5_dw_shift, l5_pw_w, l5_pw_b, l5_pw_scale, l5_pw_shift),
        (l6_dw_w, l6_dw_b, l6_dw_scale, l6_dw_shift, l6_pw_w, l6_pw_b, l6_pw_scale, l6_pw_shift),
        (l7_dw_w, l7_dw_b, l7_dw_scale, l7_dw_shift, l7_pw_w, l7_pw_b, l7_pw_scale, l7_pw_shift),
    ]
    pe_aux = jnp.stack([pe_b, pe_scale, pe_shift])
    dw_w_all = jnp.stack([lw[0] for lw in layers]).astype(jnp.bfloat16)
    dw_w_rep = jnp.broadcast_to(dw_w_all[:, :, None, :],
                                (len(layers), dw_w_all.shape[1], 16, d))
    dw_aux = jnp.stack([jnp.stack([lw[1], lw[2], lw[3]]) for lw in layers])
    pw_w_all = jnp.stack([lw[4] for lw in layers]).astype(jnp.bfloat16)
    pw_aux = jnp.stack([jnp.stack([lw[5], lw[6], lw[7]]) for lw in layers])

    return _convmixer_fused(patches, pe_w.astype(jnp.bfloat16), pe_aux,
                            dw_w_rep, dw_aux, pw_w_all, pw_aux,
                            HP=hp, WP=wp, K=kk)


# speedup vs baseline: 4.4364x; 1.1061x over previous
import functools
import math

import jax
import jax.numpy as jnp
from jax.experimental import pallas as pl
from jax.experimental.pallas import tpu as pltpu

_INV_SQRT2 = 1.0 / math.sqrt(2.0)
_VMEM_LIMIT = 64 * 1024 * 1024


def _gelu(x):
    return 0.5 * x * (1.0 + jax.lax.erf(x * _INV_SQRT2))


def _convmixer_kernel(p_ref, pe_w_ref, pe_aux_ref,
                      dw_w_ref, dw_aux_ref, pw_w_ref, pw_aux_ref,
                      o_ref, ybuf_ref, s_ref, *, HP, WP, K, PAD, L):
    HW = HP * WP
    HH = WP // 2
    D = o_ref.shape[-1]

    def write_shifts(p, row_bf16):
        zc = jnp.zeros((PAD, D), jnp.bfloat16)
        padded = jnp.concatenate([zc, row_bf16, zc], axis=0)
        for j in range(K):
            s_ref[j, p] = padded[j:j + WP, :]

    zrow = jnp.zeros((WP, D), jnp.bfloat16)
    for j in range(K):
        for r in range(PAD):
            s_ref[j, r] = zrow
            s_ref[j, HP + PAD + r] = zrow

    feat = jnp.dot(p_ref[0], pe_w_ref[...], preferred_element_type=jnp.float32)
    a = pe_aux_ref[...]
    feat = _gelu(feat + a[0]) * a[1] + a[2]
    for r in range(HP):
        write_shifts(r + PAD, feat[r * WP:(r + 1) * WP].astype(jnp.bfloat16))

    def layer(l, _):
        dbias, dscale, dshift = dw_aux_ref[l, 0], dw_aux_ref[l, 1], dw_aux_ref[l, 2]

        def half_epilogue(g, bias, scale, shift, x0):
            xg = g + bias
            e = jax.lax.erf(xg * _INV_SQRT2)
            return (xg + xg * e) * scale + (shift + x0)

        def row(h, _):
            accA = accB = None
            for i in range(K):
                for j in range(K):
                    wt = dw_w_ref[l, i * K + j]
                    if j == 0 and i % 3 == 0:
                        gA = s_ref[0, h + i, 0:HH, :] * wt
                        gB = s_ref[0, h + i, HH:WP, :] * wt
                    else:
                        gA = gA + s_ref[j, h + i, 0:HH, :] * wt
                        gB = gB + s_ref[j, h + i, HH:WP, :] * wt
                if i % 3 == 2:
                    accA = gA if accA is None else accA + gA
                    accB = gB if accB is None else accB + gB
            yA = half_epilogue(accA, dbias, dscale, dshift,
                               s_ref[PAD, h + PAD, 0:HH, :])
            yB = half_epilogue(accB, dbias, dscale, dshift,
                               s_ref[PAD, h + PAD, HH:WP, :])
            ybuf_ref[pl.ds(h * WP, HH), :] = yA
            ybuf_ref[pl.ds(h * WP + HH, HH), :] = yB
            return 0

        jax.lax.fori_loop(0, HP, row, 0)

        z = jnp.dot(ybuf_ref[...], pw_w_ref[l],
                    preferred_element_type=jnp.float32)
        pbias, pscale, pshift = pw_aux_ref[l, 0], pw_aux_ref[l, 1], pw_aux_ref[l, 2]
        for r in range(HP):
            zA = z[r * WP:r * WP + HH].astype(jnp.bfloat16)
            zB = z[r * WP + HH:(r + 1) * WP].astype(jnp.bfloat16)
            xA = zA + pbias
            xB = zB + pbias
            eA = jax.lax.erf(xA * _INV_SQRT2)
            eB = jax.lax.erf(xB * _INV_SQRT2)
            zoA = (xA + xA * eA) * pscale + pshift
            zoB = (xB + xB * eB) * pscale + pshift
            write_shifts(r + PAD, jnp.concatenate([zoA, zoB], axis=0))
        return 0

    jax.lax.fori_loop(0, L, layer, 0)

    interior = s_ref[PAD, pl.ds(PAD, HP), :, :].astype(jnp.float32)
    o_ref[0, 0, :] = jnp.mean(interior, axis=(0, 1))


def _convmixer_fused(patches, pe_w, pe_aux, dw_w, dw_aux, pw_w, pw_aux,
                     *, HP, WP, K):
    n = patches.shape[0]
    cpp = patches.shape[2]
    d = pe_w.shape[1]
    L = dw_w.shape[0]
    pad = K // 2
    hw = HP * WP

    kern = functools.partial(_convmixer_kernel, HP=HP, WP=WP, K=K, PAD=pad, L=L)
    out = pl.pallas_call(
        kern,
        out_shape=jax.ShapeDtypeStruct((n, 1, d), jnp.float32),
        grid_spec=pltpu.PrefetchScalarGridSpec(
            num_scalar_prefetch=0,
            grid=(n,),
            in_specs=[
                pl.BlockSpec((1, hw, cpp), lambda b: (b, 0, 0)),
                pl.BlockSpec((cpp, d), lambda b: (0, 0)),
                pl.BlockSpec((3, d), lambda b: (0, 0)),
                pl.BlockSpec((L, K * K, 16, d), lambda b: (0, 0, 0, 0)),
                pl.BlockSpec((L, 3, 16, d), lambda b: (0, 0, 0, 0)),
                pl.BlockSpec((L, d, d), lambda b: (0, 0, 0)),
                pl.BlockSpec((L, 3, 16, d), lambda b: (0, 0, 0, 0)),
            ],
            out_specs=pl.BlockSpec((1, 1, d), lambda b: (b, 0, 0)),
            scratch_shapes=[
                pltpu.VMEM((hw, d), jnp.bfloat16),
                pltpu.VMEM((K, HP + 2 * pad, WP, d), jnp.bfloat16),
            ],
        ),
        compiler_params=pltpu.CompilerParams(
            dimension_semantics=("parallel",),
            vmem_limit_bytes=_VMEM_LIMIT),
        cost_estimate=pl.CostEstimate(
            flops=n * hw * d * (2 * cpp + L * (2 * K * K + 2 * d)),
            transcendentals=n * hw * d * (1 + 2 * L),
            bytes_accessed=n * hw * cpp * 2 + L * d * d * 2 + n * d * 4),
    )(patches, pe_w, pe_aux, dw_w, dw_aux, pw_w, pw_aux)
    return out.reshape(n, d)


def kernel(x, pe_w, pe_b, pe_scale, pe_shift, l0_dw_w, l0_dw_b, l0_dw_scale, l0_dw_shift, l0_pw_w, l0_pw_b, l0_pw_scale, l0_pw_shift, l1_dw_w, l1_dw_b, l1_dw_scale, l1_dw_shift, l1_pw_w, l1_pw_b, l1_pw_scale, l1_pw_shift, l2_dw_w, l2_dw_b, l2_dw_scale, l2_dw_shift, l2_pw_w, l2_pw_b, l2_pw_scale, l2_pw_shift, l3_dw_w, l3_dw_b, l3_dw_scale, l3_dw_shift, l3_pw_w, l3_pw_b, l3_pw_scale, l3_pw_shift, l4_dw_w, l4_dw_b, l4_dw_scale, l4_dw_shift, l4_pw_w, l4_pw_b, l4_pw_scale, l4_pw_shift, l5_dw_w, l5_dw_b, l5_dw_scale, l5_dw_shift, l5_pw_w, l5_pw_b, l5_pw_scale, l5_pw_shift, l6_dw_w, l6_dw_b, l6_dw_scale, l6_dw_shift, l6_pw_w, l6_pw_b, l6_pw_scale, l6_pw_shift, l7_dw_w, l7_dw_b, l7_dw_scale, l7_dw_shift, l7_pw_w, l7_pw_b, l7_pw_scale, l7_pw_shift):
    n, c, h, w = x.shape
    p = 7
    hp, wp = h // p, w // p
    d = pe_w.shape[1]
    kk = int(round(math.sqrt(l0_dw_w.shape[0])))

    patches = (x.reshape(n, c, hp, p, wp, p)
                .transpose(0, 2, 4, 1, 3, 5)
                .reshape(n, hp * wp, c * p * p)
                .astype(jnp.bfloat16))

    layers = [
        (l0_dw_w, l0_dw_b, l0_dw_scale, l0_dw_shift, l0_pw_w, l0_pw_b, l0_pw_scale, l0_pw_shift),
        (l1_dw_w, l1_dw_b, l1_dw_scale, l1_dw_shift, l1_pw_w, l1_pw_b, l1_pw_scale, l1_pw_shift),
        (l2_dw_w, l2_dw_b, l2_dw_scale, l2_dw_shift, l2_pw_w, l2_pw_b, l2_pw_scale, l2_pw_shift),
        (l3_dw_w, l3_dw_b, l3_dw_scale, l3_dw_shift, l3_pw_w, l3_pw_b, l3_pw_scale, l3_pw_shift),
        (l4_dw_w, l4_dw_b, l4_dw_scale, l4_dw_shift, l4_pw_w, l4_pw_b, l4_pw_scale, l4_pw_shift),
        (l5_dw_w, l5_dw_b, l5_dw_scale, l5_dw_shift, l5_pw_w, l5_pw_b, l5_pw_scale, l5_pw_shift),
        (l6_dw_w, l6_dw_b, l6_dw_scale, l6_dw_shift, l6_pw_w, l6_pw_b, l6_pw_scale, l6_pw_shift),
        (l7_dw_w, l7_dw_b, l7_dw_scale, l7_dw_shift, l7_pw_w, l7_pw_b, l7_pw_scale, l7_pw_shift),
    ]
    pe_aux = jnp.stack([pe_b, pe_scale, pe_shift])
    dw_w_all = jnp.stack([lw[0] for lw in layers]).astype(jnp.bfloat16)
    dw_w_rep = jnp.broadcast_to(dw_w_all[:, :, None, :],
                                (len(layers), dw_w_all.shape[1], 16, d))
    dw_aux = jnp.stack(
        [jnp.stack([lw[1], 0.5 * lw[2], lw[3]]) for lw in layers]
    ).astype(jnp.bfloat16)
    dw_aux = jnp.broadcast_to(dw_aux[:, :, None, :], (len(layers), 3, 16, d))
    pw_w_all = jnp.stack([lw[4] for lw in layers]).astype(jnp.bfloat16)
    pw_aux = jnp.stack(
        [jnp.stack([lw[5], 0.5 * lw[6], lw[7]]) for lw in layers]
    ).astype(jnp.bfloat16)
    pw_aux = jnp.broadcast_to(pw_aux[:, :, None, :], (len(layers), 3, 16, d))

    return _convmixer_fused(patches, pe_w.astype(jnp.bfloat16), pe_aux,
                            dw_w_rep, dw_aux, pw_w_all, pw_aux,
                            HP=hp, WP=wp, K=kk)
